# Initial kernel scaffold; baseline (speedup 1.0000x reference)
import jax
import jax.numpy as jnp
from jax.experimental import pallas as pl


def kernel(x, pos_table, ln_gamma, ln_beta):
    raise NotImplementedError("write your pallas kernel here")


# speedup vs baseline: 3.5552x; 3.5552x over previous
import jax
import jax.numpy as jnp
from jax.experimental import pallas as pl

EPS = 1e-12
BLOCK_S = 512


def _body(x_ref, pos_ref, g_ref, b_ref, o_ref):
    h = x_ref[0] + pos_ref[...]
    mean = jnp.mean(h, axis=-1, keepdims=True)
    c = h - mean
    var = jnp.mean(c * c, axis=-1, keepdims=True)
    o_ref[0] = c * jax.lax.rsqrt(var + EPS) * g_ref[...] + b_ref[...]


def kernel(x, pos_table, ln_gamma, ln_beta):
    B, S, D = x.shape
    grid = (S // BLOCK_S, B)
    return pl.pallas_call(
        _body,
        grid=grid,
        in_specs=[
            pl.BlockSpec((1, BLOCK_S, D), lambda i, j: (j, i, 0)),
            pl.BlockSpec((BLOCK_S, D), lambda i, j: (i, 0)),
            pl.BlockSpec((D,), lambda i, j: (0,)),
            pl.BlockSpec((D,), lambda i, j: (0,)),
        ],
        out_specs=pl.BlockSpec((1, BLOCK_S, D), lambda i, j: (j, i, 0)),
        out_shape=jax.ShapeDtypeStruct((B, S, D), x.dtype),
    )(x, pos_table, ln_gamma, ln_beta)


# speedup vs baseline: 4.2157x; 1.1858x over previous
import jax
import jax.numpy as jnp
from jax.experimental import pallas as pl

EPS = 1e-12
BLOCK_S = 1024


def _body(x_ref, pos_ref, g_ref, b_ref, o_ref):
    h = x_ref[0] + pos_ref[...]
    mean = jnp.mean(h, axis=-1, keepdims=True)
    c = h - mean
    var = jnp.mean(c * c, axis=-1, keepdims=True)
    o_ref[0] = c * jax.lax.rsqrt(var + EPS) * g_ref[...] + b_ref[...]


def kernel(x, pos_table, ln_gamma, ln_beta):
    B, S, D = x.shape
    grid = (S // BLOCK_S, B)
    return pl.pallas_call(
        _body,
        grid=grid,
        in_specs=[
            pl.BlockSpec((1, BLOCK_S, D), lambda i, j: (j, i, 0)),
            pl.BlockSpec((BLOCK_S, D), lambda i, j: (i, 0)),
            pl.BlockSpec((D,), lambda i, j: (0,)),
            pl.BlockSpec((D,), lambda i, j: (0,)),
        ],
        out_specs=pl.BlockSpec((1, BLOCK_S, D), lambda i, j: (j, i, 0)),
        out_shape=jax.ShapeDtypeStruct((B, S, D), x.dtype),
    )(x, pos_table, ln_gamma, ln_beta)


# speedup vs baseline: 4.5665x; 1.0832x over previous
import jax
import jax.numpy as jnp
from jax.experimental import pallas as pl

EPS = 1e-12
BLOCK_S = 2048


def _body(x_ref, pos_ref, g_ref, b_ref, o_ref):
    h = x_ref[0] + pos_ref[...]
    mean = jnp.mean(h, axis=-1, keepdims=True)
    c = h - mean
    var = jnp.mean(c * c, axis=-1, keepdims=True)
    o_ref[0] = c * jax.lax.rsqrt(var + EPS) * g_ref[...] + b_ref[...]


def kernel(x, pos_table, ln_gamma, ln_beta):
    B, S, D = x.shape
    grid = (S // BLOCK_S, B)
    return pl.pallas_call(
        _body,
        grid=grid,
        in_specs=[
            pl.BlockSpec((1, BLOCK_S, D), lambda i, j: (j, i, 0)),
            pl.BlockSpec((BLOCK_S, D), lambda i, j: (i, 0)),
            pl.BlockSpec((D,), lambda i, j: (0,)),
            pl.BlockSpec((D,), lambda i, j: (0,)),
        ],
        out_specs=pl.BlockSpec((1, BLOCK_S, D), lambda i, j: (j, i, 0)),
        out_shape=jax.ShapeDtypeStruct((B, S, D), x.dtype),
    )(x, pos_table, ln_gamma, ln_beta)
